# baseline (speedup 1.0000x reference)
import functools

import jax
import jax.numpy as jnp
from jax import lax
from jax.experimental import pallas as pl
from jax.experimental.pallas import tpu as pltpu
from jax.experimental.pallas import tpu_sc as plsc

_NC = 2
_NS = 16
_NW = _NC * _NS
_LANES = 16


def _gather_body(e, c_gat, feat_hbm, obj_hbm, sub_hbm,
                 sub_rows_hbm, obj_rows_hbm,
                 oi_v, si_v, orows_v, srows_v,
                 gsem0, gsem1, wsem0, wsem1):
    wid = lax.axis_index("s") * _NC + lax.axis_index("c")
    ew = e // _NW
    nchunk = ew // c_gat
    nbuf = 2
    gsem = (gsem0, gsem1)
    wsem = (wsem0, wsem1)

    pltpu.sync_copy(obj_hbm.at[pl.ds(wid * ew, ew)], oi_v)
    pltpu.sync_copy(sub_hbm.at[pl.ds(wid * ew, ew)], si_v)

    def load_idx_and_fire(i, b):
        csl = pl.ds(i * c_gat, c_gat)
        pltpu.async_copy(feat_hbm.at[oi_v.at[csl]], orows_v.at[b], gsem[b])
        pltpu.async_copy(feat_hbm.at[si_v.at[csl]], srows_v.at[b], gsem[b])

    def drain_and_write(i, b):
        base = wid * ew + i * c_gat
        csl = pl.ds(i * c_gat, c_gat)
        pltpu.make_async_copy(feat_hbm.at[oi_v.at[csl]], orows_v.at[b], gsem[b]).wait()
        pltpu.make_async_copy(feat_hbm.at[si_v.at[csl]], srows_v.at[b], gsem[b]).wait()
        pltpu.async_copy(orows_v.at[b], obj_rows_hbm.at[pl.ds(base, c_gat)], wsem[b])
        pltpu.async_copy(srows_v.at[b], sub_rows_hbm.at[pl.ds(base, c_gat)], wsem[b])

    def wait_writes(i, b):
        base = wid * ew + i * c_gat
        pltpu.make_async_copy(orows_v.at[b], obj_rows_hbm.at[pl.ds(base, c_gat)], wsem[b]).wait()
        pltpu.make_async_copy(srows_v.at[b], sub_rows_hbm.at[pl.ds(base, c_gat)], wsem[b]).wait()

    load_idx_and_fire(0, 0)
    load_idx_and_fire(1, 1)

    def step(io, _):
        for b in range(nbuf):
            i = io * nbuf + b
            drain_and_write(i, b)
            wait_writes(i, b)

            @pl.when(i + nbuf < nchunk)
            def _():
                load_idx_and_fire(i + nbuf, b)
        return _

    lax.fori_loop(0, nchunk // nbuf, step, None)


def _nd_lookup(idx, nd_ref):
    nrow, ncol = nd_ref.shape
    hi = (idx // ncol)[:, None]
    lo = (idx % ncol)[:, None]
    oh_hi = (hi == lax.broadcasted_iota(jnp.int32, (idx.shape[0], nrow), 1)
             ).astype(jnp.float32)
    rowsel = jnp.dot(oh_hi, nd_ref[...], preferred_element_type=jnp.float32)
    oh_lo = lo == lax.broadcasted_iota(jnp.int32, (idx.shape[0], ncol), 1)
    return jnp.sum(jnp.where(oh_lo, rowsel, 0.0), axis=1)


def _gate_body(n, sub_ref, obj_ref, nd_ref, oid_ref, sid_ref,
               p_ref, fcb_ref, g_ref, flat_ref):
    hw = sub_ref.shape[1]
    inv = 1.0 / (4 * hw)
    x1 = sub_ref[...]
    x2 = obj_ref[...]
    hi_mask = jnp.int32(-65536)
    se = lax.bitcast_convert_type(lax.shift_left(x1, 16), jnp.float32)
    so = lax.bitcast_convert_type(x1 & hi_mask, jnp.float32)
    oe = lax.bitcast_convert_type(lax.shift_left(x2, 16), jnp.float32)
    oo = lax.bitcast_convert_type(x2 & hi_mask, jnp.float32)
    s = (jnp.sum(se, axis=1) + jnp.sum(so, axis=1)
         + jnp.sum(oe, axis=1) + jnp.sum(oo, axis=1))
    q = (jnp.sum(se * se, axis=1) + jnp.sum(so * so, axis=1)
         + jnp.sum(oe * oe, axis=1) + jnp.sum(oo * oo, axis=1))
    mu = s * inv
    var = q * inv - mu * mu
    r = lax.rsqrt(var + 1e-5)
    mu2 = mu[:, None]
    r2 = r[:, None]
    t = fcb_ref[0, 0]
    for i, x in enumerate((se, so, oe, oo)):
        gam = p_ref[i, :][None, :]
        bet = p_ref[4 + i, :][None, :]
        w = p_ref[8 + i, :][None, :]
        hx = jnp.maximum((x - mu2) * r2 * gam + bet, 0.0)
        t = t + lax.dot_general(hx, w, (((1,), (1,)), ((), ())),
                                preferred_element_type=jnp.float32)
    oid = oid_ref[...]
    sid = sid_ref[...]
    nm = (_nd_lookup(oid[:, 0], nd_ref) * _nd_lookup(sid[:, 0], nd_ref))[:, None]
    nm = jnp.where(nm > 10000.0, 0.0, nm)
    g_ref[...] = jnp.tanh(t) * nm
    flat_ref[...] = sid * n + oid


def _scatter_body(e, c_sc, flat_hbm, g_hbm, a_hbm, fl_v, g_v, sem):
    wid = lax.axis_index("s") * _NC + lax.axis_index("c")
    nchunk = (e // c_sc) // _NW
    row0 = wid * nchunk
    pltpu.sync_copy(flat_hbm.at[pl.ds(row0, nchunk)], fl_v)
    pltpu.sync_copy(g_hbm.at[pl.ds(row0, nchunk)], g_v)

    grp = min(8, nchunk)
    assert nchunk % grp == 0

    def fire(io, _):
        for jj in range(grp):
            j = io * grp + jj
            pltpu.async_copy(g_v.at[j], a_hbm.at[fl_v.at[j]], sem)
        return _

    lax.fori_loop(0, nchunk // grp, fire, None)

    def drain(io, _):
        for _jj in range(grp):
            pltpu.make_async_copy(g_v.at[0], a_hbm.at[fl_v.at[0]], sem).wait()
        return _

    lax.fori_loop(0, nchunk // grp, drain, None)


def _matmul_body(a_ref, f_ref, o_ref):
    o_ref[...] = jnp.dot(a_ref[...].astype(jnp.bfloat16),
                         f_ref[...].astype(jnp.bfloat16),
                         preferred_element_type=jnp.float32)


def kernel(inst_feature, norm_degree, aggregator_matrix, rel_pair_index,
           ln_gamma, ln_beta, fc_w, fc_b):
    n, h = inst_feature.shape
    e = rel_pair_index.shape[0]
    nslice = 4
    es = e // nslice
    c_gat = 64
    c_sc = 128
    hw = h // 2
    assert es % (_NW * c_gat) == 0 and h % 128 == 0

    obj_idx = rel_pair_index[:, 0].astype(jnp.int32)
    sub_idx = rel_pair_index[:, 1].astype(jnp.int32)
    feat_b = lax.bitcast_convert_type(
        inst_feature.astype(jnp.bfloat16).reshape(n, hw, 2), jnp.int32)
    nd_tab = norm_degree.reshape(n // 128, 128)

    mesh = plsc.VectorSubcoreMesh(core_axis_name="c", subcore_axis_name="s")

    gather_k = functools.partial(
        pl.kernel,
        out_type=(
            jax.ShapeDtypeStruct((es, hw), jnp.int32),
            jax.ShapeDtypeStruct((es, hw), jnp.int32),
        ),
        mesh=mesh,
        scratch_types=[
            pltpu.VMEM((es // _NW,), jnp.int32),
            pltpu.VMEM((es // _NW,), jnp.int32),
            pltpu.VMEM((2, c_gat, hw), jnp.int32),
            pltpu.VMEM((2, c_gat, hw), jnp.int32),
            pltpu.SemaphoreType.DMA,
            pltpu.SemaphoreType.DMA,
            pltpu.SemaphoreType.DMA,
            pltpu.SemaphoreType.DMA,
        ],
    )(functools.partial(_gather_body, es, c_gat))

    rows = []
    for arr in (ln_gamma, ln_beta, fc_w.reshape(2 * h)):
        for half in (0, h):
            rows.append(arr[half:half + h:2])
            rows.append(arr[half + 1:half + h:2])
    params = jnp.concatenate(
        [jnp.stack(rows), jnp.zeros((4, hw), jnp.float32)])

    b = 1024
    gate_k = pl.pallas_call(
        functools.partial(_gate_body, n),
        grid=(es // b,),
        in_specs=[
            pl.BlockSpec((b, hw), lambda i: (i, 0)),
            pl.BlockSpec((b, hw), lambda i: (i, 0)),
            pl.BlockSpec((32, 128), lambda i: (0, 0)),
            pl.BlockSpec((b, 1), lambda i: (i, 0)),
            pl.BlockSpec((b, 1), lambda i: (i, 0)),
            pl.BlockSpec((16, hw), lambda i: (0, 0)),
            pl.BlockSpec((1, 1), lambda i: (0, 0)),
        ],
        out_specs=[
            pl.BlockSpec((b, 1), lambda i: (i, 0)),
            pl.BlockSpec((b, 1), lambda i: (i, 0)),
        ],
        out_shape=[
            jax.ShapeDtypeStruct((es, 1), jnp.float32),
            jax.ShapeDtypeStruct((es, 1), jnp.int32),
        ],
    )

    a_ref = jax.new_ref(jnp.zeros((n * n,), jnp.float32))
    nchunk_w = (es // c_sc) // _NW
    scatter_k = functools.partial(
        pl.kernel,
        out_type=(),
        mesh=mesh,
        scratch_types=[
            pltpu.VMEM((nchunk_w, c_sc), jnp.int32),
            pltpu.VMEM((nchunk_w, c_sc), jnp.float32),
            pltpu.SemaphoreType.DMA,
        ],
    )(functools.partial(_scatter_body, es, c_sc))

    g_slices = []
    for k in range(nslice):
        oi_k = lax.slice_in_dim(obj_idx, k * es, (k + 1) * es)
        si_k = lax.slice_in_dim(sub_idx, k * es, (k + 1) * es)
        sub_rows, obj_rows = gather_k(feat_b, oi_k, si_k)
        g2d, flat2d = gate_k(
            sub_rows, obj_rows,
            nd_tab, oi_k.reshape(es, 1), si_k.reshape(es, 1),
            params, fc_b.reshape(1, 1))
        scatter_k(flat2d.reshape(es // c_sc, c_sc),
                  g2d.reshape(es // c_sc, c_sc), a_ref)
        g_slices.append(g2d.reshape(es))
    g = jnp.concatenate(g_slices)
    a_mat = a_ref[...].reshape(n, n)

    bm = 512
    mm_k = pl.pallas_call(
        _matmul_body,
        grid=(n // bm,),
        in_specs=[
            pl.BlockSpec((bm, n), lambda i: (i, 0)),
            pl.BlockSpec((n, h), lambda i: (0, 0)),
        ],
        out_specs=pl.BlockSpec((bm, h), lambda i: (i, 0)),
        out_shape=jax.ShapeDtypeStruct((n, h), jnp.float32),
    )
    aggregator_feature = mm_k(a_mat, inst_feature)
    return (aggregator_feature, g)


# speedup vs baseline: 1.0133x; 1.0133x over previous
import functools

import jax
import jax.numpy as jnp
from jax import lax
from jax.experimental import pallas as pl
from jax.experimental.pallas import tpu as pltpu
from jax.experimental.pallas import tpu_sc as plsc

_NC = 2
_NS = 16
_NW = _NC * _NS
_LANES = 16


def _gather_body(e, c_gat, feat_hbm, obj_hbm, sub_hbm,
                 sub_rows_hbm, obj_rows_hbm,
                 oi_v, si_v, orows_v, srows_v,
                 gsem0, gsem1, wsem0, wsem1):
    wid = lax.axis_index("s") * _NC + lax.axis_index("c")
    ew = e // _NW
    nchunk = ew // c_gat
    nbuf = 2
    gsem = (gsem0, gsem1)
    wsem = (wsem0, wsem1)

    pltpu.sync_copy(obj_hbm.at[pl.ds(wid * ew, ew)], oi_v)
    pltpu.sync_copy(sub_hbm.at[pl.ds(wid * ew, ew)], si_v)

    def load_idx_and_fire(i, b):
        csl = pl.ds(i * c_gat, c_gat)
        pltpu.async_copy(feat_hbm.at[oi_v.at[csl]], orows_v.at[b], gsem[b])
        pltpu.async_copy(feat_hbm.at[si_v.at[csl]], srows_v.at[b], gsem[b])

    def drain_and_write(i, b):
        base = wid * ew + i * c_gat
        csl = pl.ds(i * c_gat, c_gat)
        pltpu.make_async_copy(feat_hbm.at[oi_v.at[csl]], orows_v.at[b], gsem[b]).wait()
        pltpu.make_async_copy(feat_hbm.at[si_v.at[csl]], srows_v.at[b], gsem[b]).wait()
        pltpu.async_copy(orows_v.at[b], obj_rows_hbm.at[pl.ds(base, c_gat)], wsem[b])
        pltpu.async_copy(srows_v.at[b], sub_rows_hbm.at[pl.ds(base, c_gat)], wsem[b])

    def wait_writes(i, b):
        base = wid * ew + i * c_gat
        pltpu.make_async_copy(orows_v.at[b], obj_rows_hbm.at[pl.ds(base, c_gat)], wsem[b]).wait()
        pltpu.make_async_copy(srows_v.at[b], sub_rows_hbm.at[pl.ds(base, c_gat)], wsem[b]).wait()

    load_idx_and_fire(0, 0)
    load_idx_and_fire(1, 1)

    def step(io, _):
        for b in range(nbuf):
            i = io * nbuf + b
            drain_and_write(i, b)
            wait_writes(i, b)

            @pl.when(i + nbuf < nchunk)
            def _():
                load_idx_and_fire(i + nbuf, b)
        return _

    lax.fori_loop(0, nchunk // nbuf, step, None)


def _nd_lookup(idx, nd_ref):
    nrow, ncol = nd_ref.shape
    hi = (idx // ncol)[:, None]
    lo = (idx % ncol)[:, None]
    oh_hi = (hi == lax.broadcasted_iota(jnp.int32, (idx.shape[0], nrow), 1)
             ).astype(jnp.float32)
    rowsel = jnp.dot(oh_hi, nd_ref[...], preferred_element_type=jnp.float32)
    oh_lo = lo == lax.broadcasted_iota(jnp.int32, (idx.shape[0], ncol), 1)
    return jnp.sum(jnp.where(oh_lo, rowsel, 0.0), axis=1)


def _gate_body(n, sub_ref, obj_ref, nd_ref, oid_ref, sid_ref,
               p_ref, fcb_ref, g_ref, flat_ref):
    hw = sub_ref.shape[1]
    inv = 1.0 / (4 * hw)
    x1 = sub_ref[...]
    x2 = obj_ref[...]
    hi_mask = jnp.int32(-65536)
    se = lax.bitcast_convert_type(lax.shift_left(x1, 16), jnp.float32)
    so = lax.bitcast_convert_type(x1 & hi_mask, jnp.float32)
    oe = lax.bitcast_convert_type(lax.shift_left(x2, 16), jnp.float32)
    oo = lax.bitcast_convert_type(x2 & hi_mask, jnp.float32)
    s = (jnp.sum(se, axis=1) + jnp.sum(so, axis=1)
         + jnp.sum(oe, axis=1) + jnp.sum(oo, axis=1))
    q = (jnp.sum(se * se, axis=1) + jnp.sum(so * so, axis=1)
         + jnp.sum(oe * oe, axis=1) + jnp.sum(oo * oo, axis=1))
    mu = s * inv
    var = q * inv - mu * mu
    r = lax.rsqrt(var + 1e-5)
    mu2 = mu[:, None]
    r2 = r[:, None]
    t = fcb_ref[0, 0]
    for i, x in enumerate((se, so, oe, oo)):
        gam = p_ref[i, :][None, :]
        bet = p_ref[4 + i, :][None, :]
        w = p_ref[8 + i, :][None, :]
        hx = jnp.maximum((x - mu2) * r2 * gam + bet, 0.0)
        t = t + lax.dot_general(hx, w, (((1,), (1,)), ((), ())),
                                preferred_element_type=jnp.float32)
    oid = oid_ref[...]
    sid = sid_ref[...]
    nm = (_nd_lookup(oid[:, 0], nd_ref) * _nd_lookup(sid[:, 0], nd_ref))[:, None]
    nm = jnp.where(nm > 10000.0, 0.0, nm)
    g_ref[...] = jnp.tanh(t) * nm
    flat_ref[...] = sid * n + oid


def _scatter_body(e, c_sc, flat_hbm, g_hbm, a_hbm, fl_v, g_v, sem):
    wid = lax.axis_index("s") * _NC + lax.axis_index("c")
    nchunk = (e // c_sc) // _NW
    row0 = wid * nchunk
    pltpu.sync_copy(flat_hbm.at[pl.ds(row0, nchunk)], fl_v)
    pltpu.sync_copy(g_hbm.at[pl.ds(row0, nchunk)], g_v)

    grp = min(8, nchunk)
    assert nchunk % grp == 0

    def fire(io, _):
        for jj in range(grp):
            j = io * grp + jj
            pltpu.async_copy(g_v.at[j], a_hbm.at[fl_v.at[j]], sem)
        return _

    lax.fori_loop(0, nchunk // grp, fire, None)

    def drain(io, _):
        for _jj in range(grp):
            pltpu.make_async_copy(g_v.at[0], a_hbm.at[fl_v.at[0]], sem).wait()
        return _

    lax.fori_loop(0, nchunk // grp, drain, None)


def _matmul_body(a_ref, f_ref, o_ref):
    o_ref[...] = jnp.dot(a_ref[...].astype(jnp.bfloat16),
                         f_ref[...].astype(jnp.bfloat16),
                         preferred_element_type=jnp.float32)


def kernel(inst_feature, norm_degree, aggregator_matrix, rel_pair_index,
           ln_gamma, ln_beta, fc_w, fc_b):
    n, h = inst_feature.shape
    e = rel_pair_index.shape[0]
    nslice = 4
    es = e // nslice
    c_gat = 64
    c_sc = 128
    hw = h // 2
    assert es % (_NW * c_gat) == 0 and h % 128 == 0

    obj_idx = rel_pair_index[:, 0].astype(jnp.int32)
    sub_idx = rel_pair_index[:, 1].astype(jnp.int32)
    feat_b = lax.bitcast_convert_type(
        inst_feature.astype(jnp.bfloat16).reshape(n, hw, 2), jnp.int32)
    nd_tab = norm_degree.reshape(n // 128, 128)

    mesh = plsc.VectorSubcoreMesh(core_axis_name="c", subcore_axis_name="s")

    gather_k = functools.partial(
        pl.kernel,
        out_type=(
            jax.ShapeDtypeStruct((es, hw), jnp.int32),
            jax.ShapeDtypeStruct((es, hw), jnp.int32),
        ),
        mesh=mesh,
        scratch_types=[
            pltpu.VMEM((es // _NW,), jnp.int32),
            pltpu.VMEM((es // _NW,), jnp.int32),
            pltpu.VMEM((2, c_gat, hw), jnp.int32),
            pltpu.VMEM((2, c_gat, hw), jnp.int32),
            pltpu.SemaphoreType.DMA,
            pltpu.SemaphoreType.DMA,
            pltpu.SemaphoreType.DMA,
            pltpu.SemaphoreType.DMA,
        ],
    )(functools.partial(_gather_body, es, c_gat))

    rows = []
    for arr in (ln_gamma, ln_beta, fc_w.reshape(2 * h)):
        for half in (0, h):
            rows.append(arr[half:half + h:2])
            rows.append(arr[half + 1:half + h:2])
    params = jnp.concatenate(
        [jnp.stack(rows), jnp.zeros((4, hw), jnp.float32)])

    b = 2048
    gate_k = pl.pallas_call(
        functools.partial(_gate_body, n),
        grid=(es // b,),
        in_specs=[
            pl.BlockSpec((b, hw), lambda i: (i, 0)),
            pl.BlockSpec((b, hw), lambda i: (i, 0)),
            pl.BlockSpec((32, 128), lambda i: (0, 0)),
            pl.BlockSpec((b, 1), lambda i: (i, 0)),
            pl.BlockSpec((b, 1), lambda i: (i, 0)),
            pl.BlockSpec((16, hw), lambda i: (0, 0)),
            pl.BlockSpec((1, 1), lambda i: (0, 0)),
        ],
        out_specs=[
            pl.BlockSpec((b, 1), lambda i: (i, 0)),
            pl.BlockSpec((b, 1), lambda i: (i, 0)),
        ],
        out_shape=[
            jax.ShapeDtypeStruct((es, 1), jnp.float32),
            jax.ShapeDtypeStruct((es, 1), jnp.int32),
        ],
    )

    a_ref = jax.new_ref(jnp.zeros((n * n,), jnp.float32))
    nchunk_w = (es // c_sc) // _NW
    scatter_k = functools.partial(
        pl.kernel,
        out_type=(),
        mesh=mesh,
        scratch_types=[
            pltpu.VMEM((nchunk_w, c_sc), jnp.int32),
            pltpu.VMEM((nchunk_w, c_sc), jnp.float32),
            pltpu.SemaphoreType.DMA,
        ],
    )(functools.partial(_scatter_body, es, c_sc))

    g_slices = []
    for k in range(nslice):
        oi_k = lax.slice_in_dim(obj_idx, k * es, (k + 1) * es)
        si_k = lax.slice_in_dim(sub_idx, k * es, (k + 1) * es)
        sub_rows, obj_rows = gather_k(feat_b, oi_k, si_k)
        g2d, flat2d = gate_k(
            sub_rows, obj_rows,
            nd_tab, oi_k.reshape(es, 1), si_k.reshape(es, 1),
            params, fc_b.reshape(1, 1))
        scatter_k(flat2d.reshape(es // c_sc, c_sc),
                  g2d.reshape(es // c_sc, c_sc), a_ref)
        g_slices.append(g2d.reshape(es))
    g = jnp.concatenate(g_slices)
    a_mat = a_ref[...].reshape(n, n)

    bm = 512
    mm_k = pl.pallas_call(
        _matmul_body,
        grid=(n // bm,),
        in_specs=[
            pl.BlockSpec((bm, n), lambda i: (i, 0)),
            pl.BlockSpec((n, h), lambda i: (0, 0)),
        ],
        out_specs=pl.BlockSpec((bm, h), lambda i: (i, 0)),
        out_shape=jax.ShapeDtypeStruct((n, h), jnp.float32),
    )
    aggregator_feature = mm_k(a_mat, inst_feature)
    return (aggregator_feature, g)
